# baseline (speedup 1.0000x reference)
import functools

import numpy as np

import jax
import jax.numpy as jnp
from jax.experimental import pallas as pl
from jax.experimental.pallas import tpu as pltpu


def _interp_matrix(out_size: int, in_size: int) -> np.ndarray:
    scale = np.float32(in_size / out_size)
    o = np.arange(out_size, dtype=np.float32)
    src = np.clip((o + np.float32(0.5)) * scale - np.float32(0.5),
                  np.float32(0.0), np.float32(in_size - 1))
    lo = np.floor(src).astype(np.int32)
    hi = np.minimum(lo + 1, in_size - 1)
    frac = (src - lo.astype(np.float32)).astype(np.float32)
    rows = np.arange(out_size)
    m = np.zeros((out_size, in_size), dtype=np.float32)
    np.add.at(m, (rows, lo), np.float32(1.0) - frac)
    np.add.at(m, (rows, hi), frac)
    return m


def _resize_body(ab_ref, x_ref, o_ref, tmp_ref):
    bp, h, w = x_ref.shape
    s = o_ref.shape[1]
    xb = x_ref[...].astype(jnp.bfloat16).reshape(bp * h, w)
    tmp_ref[...] = (
        jnp.dot(xb, ab_ref[s:s + w, :], preferred_element_type=jnp.float32)
        .astype(jnp.bfloat16)
        .reshape(bp, h, -1)
    )
    a = ab_ref[:s, :h]
    for b in range(bp):
        o_ref[b] = jnp.dot(
            a, tmp_ref[b], preferred_element_type=jnp.float32
        )[:, :s]


@functools.partial(jax.jit, static_argnums=(1, 2))
def _resize_planes(x_planes: jnp.ndarray, s: int, bp: int) -> jnp.ndarray:
    nc, h, w = x_planes.shape
    np_lanes = ((s + 255) // 256) * 256
    ab_np = np.zeros((s + w, np_lanes), dtype=np.float32)
    ab_np[:s, :h] = _interp_matrix(s, h)
    ab_np[s:s + w, :s] = _interp_matrix(s, w).T
    ab = jnp.asarray(ab_np, dtype=jnp.bfloat16)
    return pl.pallas_call(
        _resize_body,
        out_shape=jax.ShapeDtypeStruct((nc, s, s), x_planes.dtype),
        grid=(nc // bp,),
        in_specs=[
            pl.BlockSpec((s + w, np_lanes), lambda i: (0, 0),
                         pipeline_mode=pl.Buffered(buffer_count=1)),
            pl.BlockSpec((bp, h, w), lambda i: (i, 0, 0)),
        ],
        out_specs=pl.BlockSpec((bp, s, s), lambda i: (i, 0, 0)),
        scratch_shapes=[pltpu.VMEM((bp, h, np_lanes), jnp.bfloat16)],
        compiler_params=pltpu.CompilerParams(
            dimension_semantics=("parallel",),
            vmem_limit_bytes=64 << 20,
        ),
    )(ab, x_planes)


def kernel(x):
    n, c, h, w = x.shape
    s = 224
    nc = n * c
    bp = 24 if nc % 24 == 0 else (8 if nc % 8 == 0 else 1)
    out = _resize_planes(x.reshape(nc, h, w), s, bp)
    return out.reshape(n, c, s, s)


# speedup vs baseline: 1.0053x; 1.0053x over previous
import functools

import jax
import jax.numpy as jnp
from jax import lax
from jax.experimental import pallas as pl
from jax.experimental.pallas import tpu as pltpu


def _interp_weights(out_size: int, in_size: int, shape, out_axis, in_axis):
    o = lax.broadcasted_iota(jnp.int32, shape, out_axis).astype(jnp.float32)
    i = lax.broadcasted_iota(jnp.int32, shape, in_axis).astype(jnp.float32)
    scale = jnp.float32(in_size / out_size)
    src = jnp.clip((o + 0.5) * scale - 0.5, 0.0, jnp.float32(in_size - 1))
    w = jnp.maximum(0.0, 1.0 - jnp.abs(i - src))
    if shape[out_axis] > out_size:
        w = jnp.where(o < out_size, w, 0.0)
    return w


def _resize_body(x_ref, o_ref, tmp_ref):
    bp, h, w = x_ref.shape
    s = o_ref.shape[1]
    np_lanes = tmp_ref.shape[2]
    bt = _interp_weights(s, w, (w, np_lanes), 1, 0).astype(jnp.bfloat16)
    a = _interp_weights(s, h, (s, h), 0, 1).astype(jnp.bfloat16)
    xb = x_ref[...].astype(jnp.bfloat16).reshape(bp * h, w)
    tmp_ref[...] = (
        jnp.dot(xb, bt, preferred_element_type=jnp.float32)
        .astype(jnp.bfloat16)
        .reshape(bp, h, -1)
    )
    for b in range(bp):
        o_ref[b] = jnp.dot(
            a, tmp_ref[b], preferred_element_type=jnp.float32
        )[:, :s]


@functools.partial(jax.jit, static_argnums=(1, 2))
def _resize_planes(x_planes: jnp.ndarray, s: int, bp: int) -> jnp.ndarray:
    nc, h, w = x_planes.shape
    np_lanes = ((s + 255) // 256) * 256
    return pl.pallas_call(
        _resize_body,
        out_shape=jax.ShapeDtypeStruct((nc, s, s), x_planes.dtype),
        grid=(nc // bp,),
        in_specs=[
            pl.BlockSpec((bp, h, w), lambda i: (i, 0, 0)),
        ],
        out_specs=pl.BlockSpec((bp, s, s), lambda i: (i, 0, 0)),
        scratch_shapes=[pltpu.VMEM((bp, h, np_lanes), jnp.bfloat16)],
        compiler_params=pltpu.CompilerParams(
            dimension_semantics=("parallel",),
            vmem_limit_bytes=64 << 20,
        ),
    )(x_planes)


def kernel(x):
    n, c, h, w = x.shape
    s = 224
    nc = n * c
    bp = 32 if nc % 32 == 0 else (8 if nc % 8 == 0 else 1)
    out = _resize_planes(x.reshape(nc, h, w), s, bp)
    return out.reshape(n, c, s, s)
